# baseline (speedup 1.0000x reference)
import jax
import jax.numpy as jnp
from jax import lax
from jax.experimental import pallas as pl
from jax.experimental.pallas import tpu as pltpu
from jax.experimental.pallas import tpu_sc as plsc

N = 10000
E = 160000
D = 256
H = 512

NPAD = 10240
CB = 128
NCB = H // CB
NCH = 1280
JCH = NCH // 16
JHALF = JCH // 2
EPAD = NCH * 128
ROWS_PER_TILE = NPAD // 16
EPS = 1e-5


def _gelu(v):
    return 0.5 * v * (1.0 + lax.erf(v * 0.7071067811865476))


def _hist_body(dsth, srch, hout, idxbuf, ones_v, zrow_v, hist_sh):
    c = lax.axis_index("c")
    s = lax.axis_index("s")
    for k in range(8):
        ones_v[pl.ds(16 * k, 16)] = jnp.ones((16,), jnp.float32)

    def zr(i, carry):
        zrow_v[pl.ds(16 * i, 16)] = jnp.zeros((16,), jnp.float32)
        return carry

    lax.fori_loop(0, ROWS_PER_TILE // 16, zr, 0)
    pltpu.sync_copy(zrow_v, hist_sh.at[pl.ds(s * ROWS_PER_TILE, ROWS_PER_TILE)])

    @pl.when(c == 0)
    def _():
        pltpu.sync_copy(dsth.at[pl.ds(s * JCH, JCH), :], idxbuf)

    @pl.when(c == 1)
    def _():
        pltpu.sync_copy(srch.at[pl.ds(s * JCH, JCH), :], idxbuf)

    plsc.subcore_barrier()

    def chunk(j, carry):
        pltpu.sync_copy(ones_v, hist_sh.at[idxbuf.at[j]], add=True)
        return carry

    lax.fori_loop(0, JCH, chunk, 0)
    plsc.subcore_barrier()
    pltpu.sync_copy(
        hist_sh.at[pl.ds(s * ROWS_PER_TILE, ROWS_PER_TILE)],
        hout.at[c, pl.ds(s * ROWS_PER_TILE, ROWS_PER_TILE)],
    )


def _sc_hist(dsth, srch):
    mesh = plsc.VectorSubcoreMesh(core_axis_name="c", subcore_axis_name="s")
    fn = pl.kernel(
        _hist_body,
        out_type=jax.ShapeDtypeStruct((2, NPAD), jnp.float32),
        mesh=mesh,
        scratch_types=[
            pltpu.VMEM((JCH, 128), jnp.int32),
            pltpu.VMEM((128,), jnp.float32),
            pltpu.VMEM((ROWS_PER_TILE,), jnp.float32),
            pltpu.VMEM_SHARED((NPAD,), jnp.float32),
        ],
    )
    return fn(dsth, srch)


def _seg_body(z0, z1, z2, z3, dstp, srcp, o0, o1, o2, o3,
              dstbuf, srcbuf, rows_a, rows_b, zb, accsh,
              sem_a1, sem_a2, sem_b1, sem_b2, sem_c, sem_d):
    c = lax.axis_index("c")
    s = lax.axis_index("s")

    def zr(i, carry):
        for k in range(8):
            zb[i, pl.ds(16 * k, 16)] = jnp.zeros((16,), jnp.float32)
        return carry

    lax.fori_loop(0, zb.shape[0], zr, 0)

    def do_block(z_ref, o_ref):
        for k in range(ROWS_PER_TILE // zb.shape[0]):
            pltpu.sync_copy(
                zb,
                accsh.at[pl.ds(s * ROWS_PER_TILE + k * zb.shape[0],
                               zb.shape[0]), :])
        plsc.subcore_barrier()

        def gather2(j, rows, s1, s2):
            pltpu.async_copy(
                z_ref.at[srcbuf.at[j, pl.ds(0, 64)]],
                rows.at[pl.ds(0, 64), :], s1)
            pltpu.async_copy(
                z_ref.at[srcbuf.at[j, pl.ds(64, 64)]],
                rows.at[pl.ds(64, 64), :], s2)

        def wait2(j, rows, s1, s2):
            pltpu.make_async_copy(
                z_ref.at[srcbuf.at[j, pl.ds(0, 64)]],
                rows.at[pl.ds(0, 64), :], s1).wait()
            pltpu.make_async_copy(
                z_ref.at[srcbuf.at[j, pl.ds(64, 64)]],
                rows.at[pl.ds(64, 64), :], s2).wait()

        for half in range(2):
            base = s * JCH + half * JHALF
            pltpu.sync_copy(dstp.at[pl.ds(base, JHALF), :], dstbuf)
            pltpu.sync_copy(srcp.at[pl.ds(base, JHALF), :], srcbuf)
            gather2(0, rows_a, sem_a1, sem_a2)

            def step(j, carry):
                @pl.when(j % 2 == 0)
                def _():
                    wait2(j, rows_a, sem_a1, sem_a2)

                    @pl.when(j + 1 < JHALF)
                    def _():
                        @pl.when(j > 0)
                        def _():
                            pltpu.make_async_copy(
                                rows_b, accsh.at[dstbuf.at[j - 1]],
                                sem_d).wait()

                        gather2(j + 1, rows_b, sem_b1, sem_b2)

                    pltpu.async_copy(
                        rows_a, accsh.at[dstbuf.at[j]], sem_c, add=True)

                @pl.when(j % 2 == 1)
                def _():
                    wait2(j, rows_b, sem_b1, sem_b2)

                    @pl.when(j + 1 < JHALF)
                    def _():
                        pltpu.make_async_copy(
                            rows_a, accsh.at[dstbuf.at[j - 1]], sem_c).wait()

                        gather2(j + 1, rows_a, sem_a1, sem_a2)

                    pltpu.async_copy(
                        rows_b, accsh.at[dstbuf.at[j]], sem_d, add=True)

                return carry

            lax.fori_loop(0, JHALF, step, 0)
            pltpu.make_async_copy(
                rows_a, accsh.at[dstbuf.at[JHALF - 2]], sem_c).wait()
            pltpu.make_async_copy(
                rows_b, accsh.at[dstbuf.at[JHALF - 1]], sem_d).wait()
        plsc.subcore_barrier()
        pltpu.sync_copy(
            accsh.at[pl.ds(s * ROWS_PER_TILE, ROWS_PER_TILE), :],
            o_ref.at[pl.ds(s * ROWS_PER_TILE, ROWS_PER_TILE), :],
        )

    @pl.when(c == 0)
    def _():
        do_block(z0, o0)
        do_block(z1, o1)

    @pl.when(c == 1)
    def _():
        do_block(z2, o2)
        do_block(z3, o3)


def _sc_segsum(zs, dstp, srcp):
    mesh = plsc.VectorSubcoreMesh(core_axis_name="c", subcore_axis_name="s")
    fn = pl.kernel(
        _seg_body,
        out_type=[jax.ShapeDtypeStruct((NPAD, CB), jnp.float32)] * NCB,
        mesh=mesh,
        scratch_types=[
            pltpu.VMEM((JHALF, 128), jnp.int32),
            pltpu.VMEM((JHALF, 128), jnp.int32),
            pltpu.VMEM((128, CB), jnp.float32),
            pltpu.VMEM((128, CB), jnp.float32),
            pltpu.VMEM((32, CB), jnp.float32),
            pltpu.VMEM_SHARED((NPAD, CB), jnp.float32),
            pltpu.SemaphoreType.DMA,
            pltpu.SemaphoreType.DMA,
            pltpu.SemaphoreType.DMA,
            pltpu.SemaphoreType.DMA,
            pltpu.SemaphoreType.DMA,
            pltpu.SemaphoreType.DMA,
        ],
    )
    return fn(zs[0], zs[1], zs[2], zs[3], dstp, srcp)


def _stats_body(x_ref, cnt_ref, out_ref):
    i = pl.program_id(0)
    xb = x_ref[...]
    cb = cnt_ref[...]
    xsq = xb * xb
    s_w = jnp.sum(xb * cb, axis=0, keepdims=True)
    s_wq = jnp.sum(xsq * cb, axis=0, keepdims=True)
    s_u = jnp.sum(xb, axis=0, keepdims=True)
    s_uq = jnp.sum(xsq, axis=0, keepdims=True)
    blk = jnp.concatenate(
        [s_w, s_wq, s_u, s_uq, jnp.zeros((4, D), jnp.float32)], axis=0)

    @pl.when(i == 0)
    def _():
        out_ref[...] = blk

    @pl.when(i > 0)
    def _():
        out_ref[...] += blk


def _tc_stats(x, cnt_src):
    rb = 1000
    return pl.pallas_call(
        _stats_body,
        grid=(N // rb,),
        in_specs=[
            pl.BlockSpec((rb, D), lambda i: (i, 0)),
            pl.BlockSpec((rb, 1), lambda i: (i, 0)),
        ],
        out_specs=pl.BlockSpec((8, D), lambda i: (0, 0)),
        out_shape=jax.ShapeDtypeStruct((8, D), jnp.float32),
    )(x, cnt_src)


def _z_body(x_ref, st_ref, w1_ref, b1_ref, g1_ref, be1_ref,
            z0_ref, z1_ref, z2_ref, z3_ref):
    inv_e = 1.0 / E
    mu = st_ref[0:1, :] * inv_e
    var = st_ref[1:2, :] * inv_e - mu * mu
    sc = g1_ref[...] * lax.rsqrt(var + EPS)
    sh = be1_ref[...] - mu * sc
    xn = x_ref[...] * sc + sh
    zz = jnp.dot(xn.astype(jnp.bfloat16), w1_ref[...].astype(jnp.bfloat16),
                 preferred_element_type=jnp.float32)
    zz = _gelu(zz + b1_ref[...])
    z0_ref[...] = zz[:, 0:128]
    z1_ref[...] = zz[:, 128:256]
    z2_ref[...] = zz[:, 256:384]
    z3_ref[...] = zz[:, 384:512]


def _tc_z(x, stats, w1, b1, g1, be1):
    rb = 1000
    zspec = pl.BlockSpec((rb, CB), lambda i: (i, 0))
    return pl.pallas_call(
        _z_body,
        grid=(N // rb,),
        in_specs=[
            pl.BlockSpec((rb, D), lambda i: (i, 0)),
            pl.BlockSpec((8, D), lambda i: (0, 0)),
            pl.BlockSpec((D, H), lambda i: (0, 0)),
            pl.BlockSpec((1, H), lambda i: (0, 0)),
            pl.BlockSpec((1, D), lambda i: (0, 0)),
            pl.BlockSpec((1, D), lambda i: (0, 0)),
        ],
        out_specs=[zspec] * NCB,
        out_shape=[jax.ShapeDtypeStruct((N, CB), jnp.float32)] * NCB,
    )(x, stats, w1, b1, g1, be1)


def _aggstats_body(s0_ref, s1_ref, s2_ref, s3_ref, cnt_ref, out_ref):
    i = pl.program_id(0)
    inv = 1.0 / jnp.maximum(cnt_ref[...], 1.0)
    sums = []
    sqs = []
    for ref in (s0_ref, s1_ref, s2_ref, s3_ref):
        aggb = ref[...] * inv
        sums.append(jnp.sum(aggb, axis=0, keepdims=True))
        sqs.append(jnp.sum(aggb * aggb, axis=0, keepdims=True))
    blk = jnp.concatenate(
        [jnp.concatenate(sums, axis=1),
         jnp.concatenate(sqs, axis=1),
         jnp.zeros((6, H), jnp.float32)], axis=0)

    @pl.when(i == 0)
    def _():
        out_ref[...] = blk

    @pl.when(i > 0)
    def _():
        out_ref[...] += blk


def _tc_aggstats(summed, cnt_dst):
    rb = 1000
    sspec = pl.BlockSpec((rb, CB), lambda i: (i, 0))
    return pl.pallas_call(
        _aggstats_body,
        grid=(N // rb,),
        in_specs=[sspec] * NCB + [pl.BlockSpec((rb, 1), lambda i: (i, 0))],
        out_specs=pl.BlockSpec((8, H), lambda i: (0, 0)),
        out_shape=jax.ShapeDtypeStruct((8, H), jnp.float32),
    )(*summed, cnt_dst)


def _final_body(x_ref, s0_ref, s1_ref, s2_ref, s3_ref, cnt_ref,
                stx_ref, sta_ref, w2_ref, b2_ref, g2_ref, be2_ref, out_ref):
    inv_n = 1.0 / N
    mu_x = stx_ref[2:3, :] * inv_n
    var_x = stx_ref[3:4, :] * inv_n - mu_x * mu_x
    sx = g2_ref[:, 0:D] * lax.rsqrt(var_x + EPS)
    tx = be2_ref[:, 0:D] - mu_x * sx
    mu_a = sta_ref[0:1, :] * inv_n
    var_a = sta_ref[1:2, :] * inv_n - mu_a * mu_a
    sa = g2_ref[:, D:D + H] * lax.rsqrt(var_a + EPS)
    ta = be2_ref[:, D:D + H] - mu_a * sa

    xn = x_ref[...] * sx + tx
    acc = jnp.dot(xn.astype(jnp.bfloat16),
                  w2_ref[0:D, :].astype(jnp.bfloat16),
                  preferred_element_type=jnp.float32)
    inv = 1.0 / jnp.maximum(cnt_ref[...], 1.0)
    for cbi, ref in enumerate((s0_ref, s1_ref, s2_ref, s3_ref)):
        lo = cbi * CB
        aggn = ref[...] * inv * sa[:, lo:lo + CB] + ta[:, lo:lo + CB]
        acc += jnp.dot(aggn.astype(jnp.bfloat16),
                       w2_ref[D + lo:D + lo + CB, :].astype(jnp.bfloat16),
                       preferred_element_type=jnp.float32)
    out_ref[...] = _gelu(acc + b2_ref[...])


def _tc_final(x, summed, cnt_dst, stats_x, stats_a, w2, b2, g2, be2):
    rb = 1000
    sspec = pl.BlockSpec((rb, CB), lambda i: (i, 0))
    return pl.pallas_call(
        _final_body,
        grid=(N // rb,),
        in_specs=[
            pl.BlockSpec((rb, D), lambda i: (i, 0)),
            sspec, sspec, sspec, sspec,
            pl.BlockSpec((rb, 1), lambda i: (i, 0)),
            pl.BlockSpec((8, D), lambda i: (0, 0)),
            pl.BlockSpec((8, H), lambda i: (0, 0)),
            pl.BlockSpec((D + H, H), lambda i: (0, 0)),
            pl.BlockSpec((1, H), lambda i: (0, 0)),
            pl.BlockSpec((1, D + H), lambda i: (0, 0)),
            pl.BlockSpec((1, D + H), lambda i: (0, 0)),
        ],
        out_specs=pl.BlockSpec((rb, H), lambda i: (i, 0)),
        out_shape=jax.ShapeDtypeStruct((N, H), jnp.float32),
    )(x, *summed, cnt_dst, stats_x, stats_a, w2, b2, g2, be2)


def kernel(x, edge_index, bn1_gamma, bn1_beta, W1, b1, bn2_gamma, bn2_beta,
           W2, b2):
    assert x.shape == (N, D) and edge_index.shape == (2, E)

    dst = edge_index[0]
    src = edge_index[1]
    pe = EPAD - E
    padh = jnp.full((pe,), NPAD - 1, jnp.int32)
    dsth = jnp.concatenate([dst, padh]).reshape(NCH, 128)
    srch = jnp.concatenate([src, padh]).reshape(NCH, 128)
    pads = (jnp.arange(pe, dtype=jnp.int32) * 64) % N
    srcp = jnp.concatenate([src, pads]).reshape(NCH, 128)

    hist = _sc_hist(dsth, srch)
    cnt_dst = hist[0].reshape(NPAD, 1)
    cnt_src = hist[1, :N].reshape(N, 1)

    stats_x = _tc_stats(x, cnt_src)
    zs = _tc_z(x, stats_x, W1, b1.reshape(1, H),
               bn1_gamma.reshape(1, D), bn1_beta.reshape(1, D))
    summed = _sc_segsum(zs, dsth, srcp)
    stats_a = _tc_aggstats(summed, cnt_dst)
    out = _tc_final(x, summed, cnt_dst, stats_x, stats_a, W2,
                    b2.reshape(1, H), bn2_gamma.reshape(1, D + H),
                    bn2_beta.reshape(1, D + H))
    return out


# speedup vs baseline: 1.0188x; 1.0188x over previous
import jax
import jax.numpy as jnp
from jax import lax
from jax.experimental import pallas as pl
from jax.experimental.pallas import tpu as pltpu
from jax.experimental.pallas import tpu_sc as plsc

N = 10000
E = 160000
D = 256
H = 512

NPAD = 10240
CB = 128
NCB = H // CB
NCH = 1280
JCH = NCH // 16
JHALF = JCH // 2
EPAD = NCH * 128
ROWS_PER_TILE = NPAD // 16
EPS = 1e-5


def _gelu(v):
    return 0.5 * v * (1.0 + lax.erf(v * 0.7071067811865476))


def _hist_body(edges, hout, idxbuf, ones_v, zrow_v, hist_sh):
    c = lax.axis_index("c")
    s = lax.axis_index("s")
    for k in range(8):
        ones_v[pl.ds(16 * k, 16)] = jnp.ones((16,), jnp.float32)

    def zr(i, carry):
        zrow_v[pl.ds(16 * i, 16)] = jnp.zeros((16,), jnp.float32)
        return carry

    lax.fori_loop(0, ROWS_PER_TILE // 16, zr, 0)
    pltpu.sync_copy(zrow_v, hist_sh.at[pl.ds(s * ROWS_PER_TILE, ROWS_PER_TILE)])
    pltpu.sync_copy(edges.at[c, pl.ds(s * JCH, JCH), :], idxbuf)
    plsc.subcore_barrier()

    def chunk(j, carry):
        pltpu.sync_copy(ones_v, hist_sh.at[idxbuf.at[j]], add=True)
        return carry

    nj = jnp.where(s == 15, JCH - (EPAD - E) // 128, JCH)
    lax.fori_loop(0, nj, chunk, 0)
    plsc.subcore_barrier()
    pltpu.sync_copy(
        hist_sh.at[pl.ds(s * ROWS_PER_TILE, ROWS_PER_TILE)],
        hout.at[c, pl.ds(s * ROWS_PER_TILE, ROWS_PER_TILE)],
    )


def _sc_hist(edges):
    mesh = plsc.VectorSubcoreMesh(core_axis_name="c", subcore_axis_name="s")
    fn = pl.kernel(
        _hist_body,
        out_type=jax.ShapeDtypeStruct((2, NPAD), jnp.float32),
        mesh=mesh,
        scratch_types=[
            pltpu.VMEM((JCH, 128), jnp.int32),
            pltpu.VMEM((128,), jnp.float32),
            pltpu.VMEM((ROWS_PER_TILE,), jnp.float32),
            pltpu.VMEM_SHARED((NPAD,), jnp.float32),
        ],
    )
    return fn(edges)


def _seg_body(z0, z1, z2, z3, edges, o0, o1, o2, o3,
              dstbuf, srcbuf, rows_a, rows_b, zb, accsh,
              sem_a1, sem_a2, sem_b1, sem_b2, sem_c, sem_d):
    c = lax.axis_index("c")
    s = lax.axis_index("s")

    def zr(i, carry):
        for k in range(8):
            zb[i, pl.ds(16 * k, 16)] = jnp.zeros((16,), jnp.float32)
        return carry

    lax.fori_loop(0, zb.shape[0], zr, 0)

    def do_block(z_ref, o_ref):
        for k in range(ROWS_PER_TILE // zb.shape[0]):
            pltpu.sync_copy(
                zb,
                accsh.at[pl.ds(s * ROWS_PER_TILE + k * zb.shape[0],
                               zb.shape[0]), :])
        plsc.subcore_barrier()

        def gather2(j, rows, s1, s2):
            pltpu.async_copy(
                z_ref.at[srcbuf.at[j, pl.ds(0, 64)]],
                rows.at[pl.ds(0, 64), :], s1)
            pltpu.async_copy(
                z_ref.at[srcbuf.at[j, pl.ds(64, 64)]],
                rows.at[pl.ds(64, 64), :], s2)

        def wait2(j, rows, s1, s2):
            pltpu.make_async_copy(
                z_ref.at[srcbuf.at[j, pl.ds(0, 64)]],
                rows.at[pl.ds(0, 64), :], s1).wait()
            pltpu.make_async_copy(
                z_ref.at[srcbuf.at[j, pl.ds(64, 64)]],
                rows.at[pl.ds(64, 64), :], s2).wait()

        for half in range(2):
            base = s * JCH + half * JHALF
            pltpu.sync_copy(edges.at[0, pl.ds(base, JHALF), :], dstbuf)
            pltpu.sync_copy(edges.at[1, pl.ds(base, JHALF), :], srcbuf)
            gather2(0, rows_a, sem_a1, sem_a2)

            def step(j, carry):
                @pl.when(j % 2 == 0)
                def _():
                    wait2(j, rows_a, sem_a1, sem_a2)

                    @pl.when(j + 1 < JHALF)
                    def _():
                        @pl.when(j > 0)
                        def _():
                            pltpu.make_async_copy(
                                rows_b, accsh.at[dstbuf.at[j - 1]],
                                sem_d).wait()

                        gather2(j + 1, rows_b, sem_b1, sem_b2)

                    pltpu.async_copy(
                        rows_a, accsh.at[dstbuf.at[j]], sem_c, add=True)

                @pl.when(j % 2 == 1)
                def _():
                    wait2(j, rows_b, sem_b1, sem_b2)

                    @pl.when(j + 1 < JHALF)
                    def _():
                        pltpu.make_async_copy(
                            rows_a, accsh.at[dstbuf.at[j - 1]], sem_c).wait()

                        gather2(j + 1, rows_a, sem_a1, sem_a2)

                    pltpu.async_copy(
                        rows_b, accsh.at[dstbuf.at[j]], sem_d, add=True)

                return carry

            lax.fori_loop(0, JHALF, step, 0)
            pltpu.make_async_copy(
                rows_a, accsh.at[dstbuf.at[JHALF - 2]], sem_c).wait()
            pltpu.make_async_copy(
                rows_b, accsh.at[dstbuf.at[JHALF - 1]], sem_d).wait()
        plsc.subcore_barrier()
        pltpu.sync_copy(
            accsh.at[pl.ds(s * ROWS_PER_TILE, ROWS_PER_TILE), :],
            o_ref.at[pl.ds(s * ROWS_PER_TILE, ROWS_PER_TILE), :],
        )

    @pl.when(c == 0)
    def _():
        do_block(z0, o0)
        do_block(z1, o1)

    @pl.when(c == 1)
    def _():
        do_block(z2, o2)
        do_block(z3, o3)


def _sc_segsum(zs, edges):
    mesh = plsc.VectorSubcoreMesh(core_axis_name="c", subcore_axis_name="s")
    fn = pl.kernel(
        _seg_body,
        out_type=[jax.ShapeDtypeStruct((NPAD, CB), jnp.float32)] * NCB,
        mesh=mesh,
        scratch_types=[
            pltpu.VMEM((JHALF, 128), jnp.int32),
            pltpu.VMEM((JHALF, 128), jnp.int32),
            pltpu.VMEM((128, CB), jnp.float32),
            pltpu.VMEM((128, CB), jnp.float32),
            pltpu.VMEM((32, CB), jnp.float32),
            pltpu.VMEM_SHARED((NPAD, CB), jnp.float32),
            pltpu.SemaphoreType.DMA,
            pltpu.SemaphoreType.DMA,
            pltpu.SemaphoreType.DMA,
            pltpu.SemaphoreType.DMA,
            pltpu.SemaphoreType.DMA,
            pltpu.SemaphoreType.DMA,
        ],
    )
    return fn(zs[0], zs[1], zs[2], zs[3], edges)


def _stats_body(x_ref, cnt_ref, out_ref):
    i = pl.program_id(0)
    xb = x_ref[...]
    cb = cnt_ref[...]
    xsq = xb * xb
    s_w = jnp.sum(xb * cb, axis=0, keepdims=True)
    s_wq = jnp.sum(xsq * cb, axis=0, keepdims=True)
    s_u = jnp.sum(xb, axis=0, keepdims=True)
    s_uq = jnp.sum(xsq, axis=0, keepdims=True)
    blk = jnp.concatenate(
        [s_w, s_wq, s_u, s_uq, jnp.zeros((4, D), jnp.float32)], axis=0)

    @pl.when(i == 0)
    def _():
        out_ref[...] = blk

    @pl.when(i > 0)
    def _():
        out_ref[...] += blk


def _tc_stats(x, cnt_src):
    rb = 1000
    return pl.pallas_call(
        _stats_body,
        grid=(N // rb,),
        in_specs=[
            pl.BlockSpec((rb, D), lambda i: (i, 0)),
            pl.BlockSpec((rb, 1), lambda i: (i, 0)),
        ],
        out_specs=pl.BlockSpec((8, D), lambda i: (0, 0)),
        out_shape=jax.ShapeDtypeStruct((8, D), jnp.float32),
    )(x, cnt_src)


def _z_body(x_ref, st_ref, w1_ref, b1_ref, g1_ref, be1_ref,
            z0_ref, z1_ref, z2_ref, z3_ref):
    inv_e = 1.0 / E
    mu = st_ref[0:1, :] * inv_e
    var = st_ref[1:2, :] * inv_e - mu * mu
    sc = g1_ref[...] * lax.rsqrt(var + EPS)
    sh = be1_ref[...] - mu * sc
    xn = x_ref[...] * sc + sh
    zz = jnp.dot(xn, w1_ref[...], preferred_element_type=jnp.float32)
    zz = _gelu(zz + b1_ref[...])
    z0_ref[...] = zz[:, 0:128]
    z1_ref[...] = zz[:, 128:256]
    z2_ref[...] = zz[:, 256:384]
    z3_ref[...] = zz[:, 384:512]


def _tc_z(x, stats, w1, b1, g1, be1):
    rb = 1000
    zspec = pl.BlockSpec((rb, CB), lambda i: (i, 0))
    return pl.pallas_call(
        _z_body,
        grid=(N // rb,),
        in_specs=[
            pl.BlockSpec((rb, D), lambda i: (i, 0)),
            pl.BlockSpec((8, D), lambda i: (0, 0)),
            pl.BlockSpec((D, H), lambda i: (0, 0)),
            pl.BlockSpec((1, H), lambda i: (0, 0)),
            pl.BlockSpec((1, D), lambda i: (0, 0)),
            pl.BlockSpec((1, D), lambda i: (0, 0)),
        ],
        out_specs=[zspec] * NCB,
        out_shape=[jax.ShapeDtypeStruct((N, CB), jnp.float32)] * NCB,
    )(x, stats, w1, b1, g1, be1)


def _aggstats_body(s0_ref, s1_ref, s2_ref, s3_ref, cnt_ref, out_ref):
    i = pl.program_id(0)
    inv = 1.0 / jnp.maximum(cnt_ref[...], 1.0)
    sums = []
    sqs = []
    for ref in (s0_ref, s1_ref, s2_ref, s3_ref):
        aggb = ref[...] * inv
        sums.append(jnp.sum(aggb, axis=0, keepdims=True))
        sqs.append(jnp.sum(aggb * aggb, axis=0, keepdims=True))
    blk = jnp.concatenate(
        [jnp.concatenate(sums, axis=1),
         jnp.concatenate(sqs, axis=1),
         jnp.zeros((6, H), jnp.float32)], axis=0)

    @pl.when(i == 0)
    def _():
        out_ref[...] = blk

    @pl.when(i > 0)
    def _():
        out_ref[...] += blk


def _tc_aggstats(summed, cnt_dst):
    rb = 1000
    sspec = pl.BlockSpec((rb, CB), lambda i: (i, 0))
    return pl.pallas_call(
        _aggstats_body,
        grid=(N // rb,),
        in_specs=[sspec] * NCB + [pl.BlockSpec((rb, 1), lambda i: (i, 0))],
        out_specs=pl.BlockSpec((8, H), lambda i: (0, 0)),
        out_shape=jax.ShapeDtypeStruct((8, H), jnp.float32),
    )(*summed, cnt_dst)


def _final_body(x_ref, s0_ref, s1_ref, s2_ref, s3_ref, cnt_ref,
                stx_ref, sta_ref, w2_ref, b2_ref, g2_ref, be2_ref, out_ref):
    inv_n = 1.0 / N
    mu_x = stx_ref[2:3, :] * inv_n
    var_x = stx_ref[3:4, :] * inv_n - mu_x * mu_x
    sx = g2_ref[:, 0:D] * lax.rsqrt(var_x + EPS)
    tx = be2_ref[:, 0:D] - mu_x * sx
    mu_a = sta_ref[0:1, :] * inv_n
    var_a = sta_ref[1:2, :] * inv_n - mu_a * mu_a
    sa = g2_ref[:, D:D + H] * lax.rsqrt(var_a + EPS)
    ta = be2_ref[:, D:D + H] - mu_a * sa

    xn = x_ref[...] * sx + tx
    acc = jnp.dot(xn, w2_ref[0:D, :], preferred_element_type=jnp.float32)
    inv = 1.0 / jnp.maximum(cnt_ref[...], 1.0)
    for cbi, ref in enumerate((s0_ref, s1_ref, s2_ref, s3_ref)):
        lo = cbi * CB
        aggn = ref[...] * inv * sa[:, lo:lo + CB] + ta[:, lo:lo + CB]
        acc += jnp.dot(aggn, w2_ref[D + lo:D + lo + CB, :],
                       preferred_element_type=jnp.float32)
    out_ref[...] = _gelu(acc + b2_ref[...])


def _tc_final(x, summed, cnt_dst, stats_x, stats_a, w2, b2, g2, be2):
    rb = 1000
    sspec = pl.BlockSpec((rb, CB), lambda i: (i, 0))
    return pl.pallas_call(
        _final_body,
        grid=(N // rb,),
        in_specs=[
            pl.BlockSpec((rb, D), lambda i: (i, 0)),
            sspec, sspec, sspec, sspec,
            pl.BlockSpec((rb, 1), lambda i: (i, 0)),
            pl.BlockSpec((8, D), lambda i: (0, 0)),
            pl.BlockSpec((8, H), lambda i: (0, 0)),
            pl.BlockSpec((D + H, H), lambda i: (0, 0)),
            pl.BlockSpec((1, H), lambda i: (0, 0)),
            pl.BlockSpec((1, D + H), lambda i: (0, 0)),
            pl.BlockSpec((1, D + H), lambda i: (0, 0)),
        ],
        out_specs=pl.BlockSpec((rb, H), lambda i: (i, 0)),
        out_shape=jax.ShapeDtypeStruct((N, H), jnp.float32),
    )(x, *summed, cnt_dst, stats_x, stats_a, w2, b2, g2, be2)


def kernel(x, edge_index, bn1_gamma, bn1_beta, W1, b1, bn2_gamma, bn2_beta,
           W2, b2):
    assert x.shape == (N, D) and edge_index.shape == (2, E)

    pe = EPAD - E
    pad = jnp.stack([jnp.full((pe,), NPAD - 1, jnp.int32),
                     (jnp.arange(pe, dtype=jnp.int32) * 64) % N])
    edges = jnp.concatenate([edge_index, pad], axis=1).reshape(2, NCH, 128)

    hist = _sc_hist(edges)
    cnt_dst = hist[0].reshape(NPAD, 1)
    cnt_src = hist[1, :N].reshape(N, 1)

    stats_x = _tc_stats(x, cnt_src)
    zs = _tc_z(x, stats_x, W1, b1.reshape(1, H),
               bn1_gamma.reshape(1, D), bn1_beta.reshape(1, D))
    summed = _sc_segsum(zs, edges)
    stats_a = _tc_aggstats(summed, cnt_dst)
    out = _tc_final(x, summed, cnt_dst, stats_x, stats_a, W2,
                    b2.reshape(1, H), bn2_gamma.reshape(1, D + H),
                    bn2_beta.reshape(1, D + H))
    return out
